# baseline (speedup 1.0000x reference)
import functools

import jax
import jax.numpy as jnp
from jax import lax
from jax.experimental import pallas as pl
from jax.experimental.pallas import tpu as pltpu
from jax.experimental.pallas import tpu_sc as plsc

_LANES = 16
_NWORK = 32
_RBLK = 2048


def _matvec_body(z_ref, w_ref, b_ref, out_ref):
    boost = b_ref[0, 0] / _LANES
    w = w_ref[...]
    for j in range(_RBLK // 128):
        zsub = z_ref[pl.ds(j * 128, 128), :]
        r = lax.dot_general(w, zsub, (((1,), (1,)), ((), ())),
                            preferred_element_type=jnp.float32)
        out_ref[pl.ds(j, 1), :] = r + boost


def _row_dot_full(z, W, b):
    n, d = z.shape
    nstep = -(-n // _RBLK)
    nrow = nstep * _RBLK // 128
    return pl.pallas_call(
        _matvec_body,
        grid=(nstep,),
        in_specs=[
            pl.BlockSpec((_RBLK, d), lambda i: (i, 0)),
            pl.BlockSpec((1, d), lambda i: (0, 0)),
            pl.BlockSpec((1, 1), lambda i: (0, 0)),
        ],
        out_specs=pl.BlockSpec((_RBLK // 128, 128), lambda i: (i, 0)),
        out_shape=jax.ShapeDtypeStruct((nrow, 128), jnp.float32),
    )(z, W, b.reshape(1, 1))


@functools.lru_cache(maxsize=None)
def _make_sc_gather(n_rows, n_nbr, s_len):
    assert n_rows % _LANES == 0
    groups = n_rows // _LANES
    q, rem = divmod(groups, _NWORK)
    gmax = q + (1 if rem else 0)
    rows_base = q * _LANES

    mesh = plsc.VectorSubcoreMesh(core_axis_name="c", subcore_axis_name="s")

    @functools.partial(
        pl.kernel,
        out_type=jax.ShapeDtypeStruct((n_rows,), jnp.float32),
        mesh=mesh,
        compiler_params=pltpu.CompilerParams(needs_layout_passes=False),
        scratch_types=[
            pltpu.VMEM((s_len + 8,), jnp.float32),
            pltpu.VMEM((gmax * _LANES, n_nbr), jnp.int32),
            pltpu.VMEM((gmax * _LANES,), jnp.float32),
            pltpu.VMEM((_LANES,), jnp.float32),
            pltpu.SemaphoreType.DMA,
        ],
    )
    def sc_gather(s_hbm, nbr_hbm, b_hbm, out_hbm, s_v, nbr_v, out_v, b_v, sem):
        nc = mesh.num_cores
        w = lax.axis_index("s") * nc + lax.axis_index("c")
        has_extra = w < rem
        ng = jnp.where(has_extra, q + 1, q)
        base_g = q * w + jnp.minimum(w, rem)
        row0 = base_g * _LANES

        pltpu.sync_copy(b_hbm, b_v)
        s_v[pl.ds(0, _LANES)] = b_v[...] * (1.0 / _LANES)

        cps = [
            pltpu.async_copy(s_hbm, s_v.at[pl.ds(8, s_len)], sem),
            pltpu.async_copy(nbr_hbm.at[pl.ds(row0, rows_base)],
                             nbr_v.at[pl.ds(0, rows_base)], sem),
        ]

        @pl.when(has_extra)
        def _():
            pltpu.async_copy(nbr_hbm.at[pl.ds(row0 + rows_base, _LANES)],
                             nbr_v.at[pl.ds(rows_base, _LANES)], sem).wait()

        for cp in cps:
            cp.wait()

        lanevec = lax.iota(jnp.int32, _LANES)

        def group_body(k, _):
            @pl.when(k < ng)
            def _():
                rowvec = lanevec + k * _LANES
                acc = jnp.zeros((_LANES,), jnp.float32)
                for j in range(n_nbr):
                    jvec = jnp.full((_LANES,), j, jnp.int32)
                    nidx = plsc.load_gather(nbr_v, [rowvec, jvec])
                    acc = acc + plsc.load_gather(s_v, [nidx + 7])
                out_v[pl.ds(k * _LANES, _LANES)] = jnp.maximum(acc, 0.0)
            return 0

        lax.fori_loop(0, gmax, group_body, 0)

        pltpu.sync_copy(out_v.at[pl.ds(0, rows_base)],
                        out_hbm.at[pl.ds(row0, rows_base)])

        @pl.when(has_extra)
        def _():
            pltpu.sync_copy(out_v.at[pl.ds(rows_base, _LANES)],
                            out_hbm.at[pl.ds(row0 + rows_base, _LANES)])

    return sc_gather


def kernel(z, neighbor, W, b):
    n, d = z.shape
    s2d = _row_dot_full(z, W, b)
    s_flat = s2d.reshape(-1)
    b16 = jnp.broadcast_to(b.astype(jnp.float32), (_LANES,))
    sc = _make_sc_gather(neighbor.shape[0], neighbor.shape[1], s_flat.shape[0])
    return sc(s_flat, neighbor, b16)


# speedup vs baseline: 1.0496x; 1.0496x over previous
import functools

import jax
import jax.numpy as jnp
from jax import lax
from jax.experimental import pallas as pl
from jax.experimental.pallas import tpu as pltpu
from jax.experimental.pallas import tpu_sc as plsc

_LANES = 16
_NWORK = 32
_RBLK = 2048


def _matvec_body(z_ref, w_ref, b_ref, out_ref):
    boost = b_ref[0, 0] / _LANES
    w = w_ref[...]
    for j in range(_RBLK // 128):
        zsub = z_ref[pl.ds(j * 128, 128), :]
        r = lax.dot_general(w, zsub, (((1,), (1,)), ((), ())),
                            preferred_element_type=jnp.float32)
        out_ref[pl.ds(j, 1), :] = r + boost


def _row_dot_full(z, W, b):
    n, d = z.shape
    nstep = -(-n // _RBLK)
    nrow = nstep * _RBLK // 128
    return pl.pallas_call(
        _matvec_body,
        grid=(nstep,),
        in_specs=[
            pl.BlockSpec((_RBLK, d), lambda i: (i, 0)),
            pl.BlockSpec((1, d), lambda i: (0, 0)),
            pl.BlockSpec((1, 1), lambda i: (0, 0)),
        ],
        out_specs=pl.BlockSpec((_RBLK // 128, 128), lambda i: (i, 0)),
        out_shape=jax.ShapeDtypeStruct((nrow, 128), jnp.float32),
    )(z, W, b.reshape(1, 1))


@functools.lru_cache(maxsize=None)
def _make_sc_gather(n_rows, n_nbr, s_len):
    assert n_rows % _LANES == 0
    groups = n_rows // _LANES
    q, rem = divmod(groups, _NWORK)
    gmax = q + (1 if rem else 0)
    rows_base = q * _LANES

    mesh = plsc.VectorSubcoreMesh(core_axis_name="c", subcore_axis_name="s")

    @functools.partial(
        pl.kernel,
        out_type=jax.ShapeDtypeStruct((n_rows,), jnp.float32),
        mesh=mesh,
        compiler_params=pltpu.CompilerParams(needs_layout_passes=False, skip_device_barrier=True, disable_bounds_checks=True, disable_semaphore_checks=True),
        scratch_types=[
            pltpu.VMEM((s_len + 8,), jnp.float32),
            pltpu.VMEM((gmax * _LANES * n_nbr,), jnp.int32),
            pltpu.VMEM((gmax * _LANES,), jnp.float32),
            pltpu.VMEM((_LANES,), jnp.float32),
            pltpu.SemaphoreType.DMA,
        ],
    )
    def sc_gather(s_hbm, nbr_hbm, b_hbm, out_hbm, s_v, nbr_v, out_v, b_v, sem):
        nc = mesh.num_cores
        w = lax.axis_index("s") * nc + lax.axis_index("c")
        has_extra = w < rem
        ng = jnp.where(has_extra, q + 1, q)
        base_g = q * w + jnp.minimum(w, rem)
        idx0 = base_g * _LANES * n_nbr
        nbase = rows_base * n_nbr
        row0 = base_g * _LANES

        pltpu.sync_copy(b_hbm, b_v)
        s_v[pl.ds(0, _LANES)] = b_v[...] * (1.0 / _LANES)

        cps = [
            pltpu.async_copy(s_hbm, s_v.at[pl.ds(8, s_len)], sem),
            pltpu.async_copy(nbr_hbm.at[pl.ds(idx0, nbase)],
                             nbr_v.at[pl.ds(0, nbase)], sem),
        ]

        @pl.when(has_extra)
        def _():
            pltpu.async_copy(nbr_hbm.at[pl.ds(idx0 + nbase, _LANES * n_nbr)],
                             nbr_v.at[pl.ds(nbase, _LANES * n_nbr)], sem).wait()

        for cp in cps:
            cp.wait()

        lanevec = lax.iota(jnp.int32, _LANES) * n_nbr

        def group_body(k, _):
            @pl.when(k < ng)
            def _():
                kbase = k * (_LANES * n_nbr)
                acc = jnp.zeros((_LANES,), jnp.float32)
                for j in range(n_nbr):
                    nidx = plsc.load_gather(nbr_v, [lanevec + (kbase + j)])
                    acc = acc + plsc.load_gather(s_v, [nidx + 7])
                out_v[pl.ds(k * _LANES, _LANES)] = jnp.maximum(acc, 0.0)
            return 0

        lax.fori_loop(0, gmax, group_body, 0)

        pltpu.sync_copy(out_v.at[pl.ds(0, rows_base)],
                        out_hbm.at[pl.ds(row0, rows_base)])

        @pl.when(has_extra)
        def _():
            pltpu.sync_copy(out_v.at[pl.ds(rows_base, _LANES)],
                            out_hbm.at[pl.ds(row0 + rows_base, _LANES)])

    return sc_gather


def kernel(z, neighbor, W, b):
    n, d = z.shape
    s2d = _row_dot_full(z, W, b)
    s_flat = s2d.reshape(-1)
    b16 = jnp.broadcast_to(b.astype(jnp.float32), (_LANES,))
    sc = _make_sc_gather(neighbor.shape[0], neighbor.shape[1], s_flat.shape[0])
    return sc(s_flat, neighbor.reshape(-1), b16)


# speedup vs baseline: 1.0539x; 1.0041x over previous
import functools

import jax
import jax.numpy as jnp
from jax import lax
from jax.experimental import pallas as pl
from jax.experimental.pallas import tpu as pltpu
from jax.experimental.pallas import tpu_sc as plsc

_LANES = 16
_NWORK = 32
_RBLK = 2048


def _matvec_body(z_ref, w_ref, b_ref, out_ref):
    boost = b_ref[0, 0] / _LANES
    w = w_ref[...]
    for j in range(_RBLK // 128):
        zsub = z_ref[pl.ds(j * 128, 128), :]
        r = lax.dot_general(w, zsub, (((1,), (1,)), ((), ())),
                            preferred_element_type=jnp.float32)
        out_ref[pl.ds(j, 1), :] = r + boost


def _row_dot_full(z, W, b):
    n, d = z.shape
    nstep = -(-n // _RBLK)
    nrow = nstep * _RBLK // 128
    return pl.pallas_call(
        _matvec_body,
        grid=(nstep,),
        in_specs=[
            pl.BlockSpec((_RBLK, d), lambda i: (i, 0)),
            pl.BlockSpec((1, d), lambda i: (0, 0)),
            pl.BlockSpec((1, 1), lambda i: (0, 0)),
        ],
        out_specs=pl.BlockSpec((_RBLK // 128, 128), lambda i: (i, 0)),
        out_shape=jax.ShapeDtypeStruct((nrow, 128), jnp.float32),
    )(z, W, b.reshape(1, 1))


@functools.lru_cache(maxsize=None)
def _make_sc_gather(n_rows, n_nbr, s_len):
    assert n_rows % _LANES == 0
    groups = n_rows // _LANES
    q, rem = divmod(groups, _NWORK)
    gmax = q + (1 if rem else 0)
    rows_base = q * _LANES

    mesh = plsc.VectorSubcoreMesh(core_axis_name="c", subcore_axis_name="s")

    @functools.partial(
        pl.kernel,
        out_type=jax.ShapeDtypeStruct((n_rows,), jnp.float32),
        mesh=mesh,
        compiler_params=pltpu.CompilerParams(needs_layout_passes=False),
        scratch_types=[
            pltpu.VMEM((s_len + 8,), jnp.float32),
            pltpu.VMEM((gmax * _LANES * n_nbr,), jnp.int32),
            pltpu.VMEM((gmax * _LANES,), jnp.float32),
            pltpu.VMEM((_LANES,), jnp.float32),
            pltpu.SemaphoreType.DMA,
        ],
    )
    def sc_gather(s_hbm, nbr_hbm, b_hbm, out_hbm, s_v, nbr_v, out_v, b_v, sem):
        nc = mesh.num_cores
        w = lax.axis_index("s") * nc + lax.axis_index("c")
        has_extra = w < rem
        ng = jnp.where(has_extra, q + 1, q)
        base_g = q * w + jnp.minimum(w, rem)
        idx0 = base_g * _LANES * n_nbr
        nbase = rows_base * n_nbr
        row0 = base_g * _LANES

        pltpu.sync_copy(b_hbm, b_v)
        s_v[pl.ds(0, _LANES)] = b_v[...] * (1.0 / _LANES)

        scp = pltpu.async_copy(s_hbm, s_v.at[pl.ds(8, s_len)], sem)
        nfull = nbase + _LANES * n_nbr

        @pl.when(has_extra)
        def _():
            pltpu.async_copy(nbr_hbm.at[pl.ds(idx0, nfull)],
                             nbr_v.at[pl.ds(0, nfull)], sem).wait()

        @pl.when(jnp.logical_not(has_extra))
        def _():
            pltpu.async_copy(nbr_hbm.at[pl.ds(idx0, nbase)],
                             nbr_v.at[pl.ds(0, nbase)], sem).wait()

        scp.wait()

        lanevec = lax.iota(jnp.int32, _LANES) * n_nbr

        def group_body(k, _):
            @pl.when(k < ng)
            def _():
                kbase = k * (_LANES * n_nbr)
                acc0 = jnp.zeros((_LANES,), jnp.float32)
                acc1 = jnp.zeros((_LANES,), jnp.float32)
                for j in range(0, n_nbr, 2):
                    nidx0 = plsc.load_gather(nbr_v, [lanevec + (kbase + j)])
                    nidx1 = plsc.load_gather(nbr_v, [lanevec + (kbase + j + 1)])
                    acc0 = acc0 + plsc.load_gather(s_v, [nidx0 + 7])
                    acc1 = acc1 + plsc.load_gather(s_v, [nidx1 + 7])
                out_v[pl.ds(k * _LANES, _LANES)] = jnp.maximum(acc0 + acc1, 0.0)
            return 0

        lax.fori_loop(0, gmax, group_body, 0)

        @pl.when(has_extra)
        def _():
            pltpu.sync_copy(out_v.at[pl.ds(0, rows_base + _LANES)],
                            out_hbm.at[pl.ds(row0, rows_base + _LANES)])

        @pl.when(jnp.logical_not(has_extra))
        def _():
            pltpu.sync_copy(out_v.at[pl.ds(0, rows_base)],
                            out_hbm.at[pl.ds(row0, rows_base)])

    return sc_gather


def kernel(z, neighbor, W, b):
    n, d = z.shape
    s2d = _row_dot_full(z, W, b)
    s_flat = s2d.reshape(-1)
    b16 = jnp.broadcast_to(b.astype(jnp.float32), (_LANES,))
    sc = _make_sc_gather(neighbor.shape[0], neighbor.shape[1], s_flat.shape[0])
    return sc(s_flat, neighbor.reshape(-1), b16)
